# baseline (speedup 1.0000x reference)
import functools
import jax
import jax.numpy as jnp
from jax import lax
from jax.experimental import pallas as pl
from jax.experimental.pallas import tpu as pltpu
from jax.experimental.pallas import tpu_sc as plsc

_K = 32
_ALPHA = 0.1


def _make_sc_mse(n):
    info = plsc.get_sparse_core_info()
    nw = info.num_cores * info.num_subcores
    lanes = info.num_lanes
    per_w = n // nw
    n_chunks = per_w // lanes
    mesh = plsc.VectorSubcoreMesh(core_axis_name="c", subcore_axis_name="s")

    @functools.partial(
        pl.kernel,
        mesh=mesh,
        out_type=jax.ShapeDtypeStruct((nw, lanes), jnp.float32),
        scratch_types=[
            pltpu.VMEM((per_w,), jnp.float32),
            pltpu.VMEM((per_w,), jnp.float32),
            pltpu.VMEM((lanes,), jnp.float32),
        ],
    )
    def sc_mse(x_hbm, y_hbm, out_hbm, xv, yv, res_v):
        wid = lax.axis_index("s") * info.num_cores + lax.axis_index("c")
        base = wid * per_w
        pltpu.sync_copy(x_hbm.at[pl.ds(base, per_w)], xv)
        pltpu.sync_copy(y_hbm.at[pl.ds(base, per_w)], yv)

        def body(i, acc):
            d = xv[pl.ds(i * lanes, lanes)] - yv[pl.ds(i * lanes, lanes)]
            return acc + d * d

        acc = jax.lax.fori_loop(0, n_chunks, body, jnp.zeros((lanes,), jnp.float32))
        res_v[...] = acc
        pltpu.sync_copy(res_v, out_hbm.at[wid])

    return sc_mse


def _topk_kernel(a_ref, bot_ref, *, n_iter, n_split):
    a = a_ref[...]
    ai = jax.lax.bitcast_convert_type(a, jnp.int32)

    r = a.shape[0]
    rs = r // n_split
    parts = [ai[i * rs:(i + 1) * rs] for i in range(n_split)]

    lo0 = jnp.full((rs, 1), -1, dtype=jnp.int32)
    hi0 = jnp.full((rs, 1), 0x7F800000, dtype=jnp.int32)

    def body(_, carry):
        out = []
        for (lo, hi), part in zip(carry, parts):
            mid = (lo + hi) >> 1
            c = jnp.sum((part > mid).astype(jnp.float32), axis=1, keepdims=True)
            take = c >= _K
            lo = jnp.where(take, mid, lo)
            hi = jnp.where(take, hi, mid)
            out.append((lo, hi))
        return tuple(out)

    carry0 = tuple((lo0, hi0) for _ in range(n_split))
    carry = jax.lax.fori_loop(0, n_iter, body, carry0)
    hi = jnp.concatenate([h for (_, h) in carry], axis=0)

    t = jax.lax.bitcast_convert_type(hi, jnp.float32)
    m = ai > hi
    sq = a * a
    c_gt = jnp.sum(m.astype(jnp.float32), axis=1, keepdims=True)
    s_le = jnp.sum(jnp.where(m, 0.0, sq), axis=1, keepdims=True)
    bot = s_le - (_K - c_gt) * (t * t)
    bot_ref[...] = jnp.sum(bot).reshape(1, 1, 1)


def kernel(x, y, attn):
    s = attn.shape[-1]
    rows = attn.size // s
    a2 = attn.reshape(rows, s)

    block_r = min(512, rows)
    grid = rows // block_r

    bot = pl.pallas_call(
        functools.partial(_topk_kernel, n_iter=31, n_split=2),
        grid=(grid,),
        in_specs=[pl.BlockSpec((block_r, s), lambda i: (i, 0))],
        out_specs=pl.BlockSpec((1, 1, 1), lambda i: (i, 0, 0)),
        out_shape=jax.ShapeDtypeStruct((grid, 1, 1), jnp.float32),
    )(a2)

    xf = x.reshape(-1)
    yf = y.reshape(-1)
    sse_parts = _make_sc_mse(xf.size)(xf, yf)

    rec_loss = jnp.sum(sse_parts) / x.size
    attn_loss = jnp.sum(bot) / attn.size
    return rec_loss + _ALPHA * attn_loss


# speedup vs baseline: 1.0002x; 1.0002x over previous
import functools
import jax
import jax.numpy as jnp
from jax import lax
from jax.experimental import pallas as pl
from jax.experimental.pallas import tpu as pltpu
from jax.experimental.pallas import tpu_sc as plsc

_K = 32
_ALPHA = 0.1


def _make_sc_mse(n):
    info = plsc.get_sparse_core_info()
    nw = info.num_cores * info.num_subcores
    lanes = info.num_lanes
    per_w = n // nw
    n_chunks = per_w // lanes
    mesh = plsc.VectorSubcoreMesh(core_axis_name="c", subcore_axis_name="s")

    @functools.partial(
        pl.kernel,
        mesh=mesh,
        out_type=jax.ShapeDtypeStruct((nw, lanes), jnp.float32),
        scratch_types=[
            pltpu.VMEM((per_w,), jnp.float32),
            pltpu.VMEM((per_w,), jnp.float32),
            pltpu.VMEM((lanes,), jnp.float32),
        ],
    )
    def sc_mse(x_hbm, y_hbm, out_hbm, xv, yv, res_v):
        wid = lax.axis_index("s") * info.num_cores + lax.axis_index("c")
        base = wid * per_w
        pltpu.sync_copy(x_hbm.at[pl.ds(base, per_w)], xv)
        pltpu.sync_copy(y_hbm.at[pl.ds(base, per_w)], yv)

        def body(i, acc):
            d = xv[pl.ds(i * lanes, lanes)] - yv[pl.ds(i * lanes, lanes)]
            return acc + d * d

        acc = jax.lax.fori_loop(0, n_chunks, body, jnp.zeros((lanes,), jnp.float32))
        res_v[...] = acc
        pltpu.sync_copy(res_v, out_hbm.at[wid])

    return sc_mse


def _topk_kernel(a_ref, bot_ref, *, n_iter, n_split):
    a = a_ref[...]
    ai = jax.lax.bitcast_convert_type(a, jnp.int32)

    r = a.shape[0]
    rs = r // n_split
    parts = [ai[i * rs:(i + 1) * rs] for i in range(n_split)]

    lo0 = jnp.full((rs, 1), -1, dtype=jnp.int32)
    hi0 = jnp.full((rs, 1), 0x7F800000, dtype=jnp.int32)

    def body(_, carry):
        out = []
        for (lo, hi), part in zip(carry, parts):
            mid = (lo + hi) >> 1
            c = jnp.sum((part > mid).astype(jnp.float32), axis=1, keepdims=True)
            take = c >= _K
            lo = jnp.where(take, mid, lo)
            hi = jnp.where(take, hi, mid)
            out.append((lo, hi))
        return tuple(out)

    carry0 = tuple((lo0, hi0) for _ in range(n_split))
    carry = jax.lax.fori_loop(0, n_iter, body, carry0)
    hi = jnp.concatenate([h for (_, h) in carry], axis=0)

    t = jax.lax.bitcast_convert_type(hi, jnp.float32)
    m = ai > hi
    sq = a * a
    c_gt = jnp.sum(m.astype(jnp.float32), axis=1, keepdims=True)
    s_le = jnp.sum(jnp.where(m, 0.0, sq), axis=1, keepdims=True)
    bot = s_le - (_K - c_gt) * (t * t)
    bot_ref[...] = jnp.sum(bot).reshape(1, 1, 1)


def kernel(x, y, attn):
    s = attn.shape[-1]
    rows = attn.size // s
    a2 = attn.reshape(rows, s)

    block_r = min(512, rows)
    grid = rows // block_r

    xf = x.reshape(-1)
    yf = y.reshape(-1)
    sse_parts = _make_sc_mse(xf.size)(xf, yf)

    bot = pl.pallas_call(
        functools.partial(_topk_kernel, n_iter=31, n_split=2),
        grid=(grid,),
        in_specs=[pl.BlockSpec((block_r, s), lambda i: (i, 0))],
        out_specs=pl.BlockSpec((1, 1, 1), lambda i: (i, 0, 0)),
        out_shape=jax.ShapeDtypeStruct((grid, 1, 1), jnp.float32),
    )(a2)

    rec_loss = jnp.sum(sse_parts) / x.size
    attn_loss = jnp.sum(bot) / attn.size
    return rec_loss + _ALPHA * attn_loss


# speedup vs baseline: 1.1003x; 1.1001x over previous
import functools
import jax
import jax.numpy as jnp
from jax.experimental import pallas as pl

_K = 32
_ALPHA = 0.1


def _mse_kernel(x_ref, y_ref, o_ref):
    d = x_ref[...] - y_ref[...]
    o_ref[...] = jnp.sum(d * d).reshape(1, 1)


def _topk_kernel(a_ref, bot_ref, *, n_iter, n_split):
    a = a_ref[...]
    r = a.shape[0]
    s = a.shape[1]
    ai = jax.lax.bitcast_convert_type(a, jnp.int32)

    rowmax = None
    big_m = None
    for c in range(s // 128):
        x = a[:, c * 128:(c + 1) * 128]
        m1 = jnp.max(x, axis=1, keepdims=True)
        m2 = jnp.max(jnp.where(x == m1, -1.0, x), axis=1, keepdims=True)
        m2 = jnp.maximum(m2, 0.0)
        rowmax = m1 if rowmax is None else jnp.maximum(rowmax, m1)
        big_m = m2 if big_m is None else jnp.minimum(big_m, m2)

    lo_all = jax.lax.bitcast_convert_type(big_m, jnp.int32) - 1
    hi_all = jax.lax.bitcast_convert_type(rowmax, jnp.int32)

    rs = r // n_split
    parts = [ai[i * rs:(i + 1) * rs] for i in range(n_split)]
    carry0 = tuple(
        (lo_all[i * rs:(i + 1) * rs], hi_all[i * rs:(i + 1) * rs])
        for i in range(n_split)
    )

    def cond(carry):
        done = jnp.array(True)
        for lo, hi in carry:
            done = jnp.logical_and(done, jnp.all(hi - lo <= 1))
        return jnp.logical_not(done)

    def body(carry):
        out = []
        for (lo, hi), part in zip(carry, parts):
            mid = (lo + hi) >> 1
            c = jnp.sum((part > mid).astype(jnp.float32), axis=1, keepdims=True)
            take = c >= _K
            lo = jnp.where(take, mid, lo)
            hi = jnp.where(take, hi, mid)
            out.append((lo, hi))
        return tuple(out)

    carry = jax.lax.while_loop(cond, body, carry0)
    hi = jnp.concatenate([h for (_, h) in carry], axis=0)

    t = jax.lax.bitcast_convert_type(hi, jnp.float32)
    m = ai > hi
    sq = a * a
    c_gt = jnp.sum(m.astype(jnp.float32), axis=1, keepdims=True)
    s_le = jnp.sum(jnp.where(m, 0.0, sq), axis=1, keepdims=True)
    bot = s_le - (_K - c_gt) * (t * t)
    bot_ref[...] = jnp.sum(bot).reshape(1, 1, 1)


def kernel(x, y, attn):
    s = attn.shape[-1]
    rows = attn.size // s
    a2 = attn.reshape(rows, s)

    block_r = min(512, rows)
    grid = rows // block_r

    bot = pl.pallas_call(
        functools.partial(_topk_kernel, n_iter=31, n_split=2),
        grid=(grid,),
        in_specs=[pl.BlockSpec((block_r, s), lambda i: (i, 0))],
        out_specs=pl.BlockSpec((1, 1, 1), lambda i: (i, 0, 0)),
        out_shape=jax.ShapeDtypeStruct((grid, 1, 1), jnp.float32),
    )(a2)

    x2 = x.reshape(-1, x.shape[-1])
    y2 = y.reshape(-1, y.shape[-1])
    sse = pl.pallas_call(
        _mse_kernel,
        out_specs=pl.BlockSpec((1, 1), lambda: (0, 0)),
        out_shape=jax.ShapeDtypeStruct((1, 1), jnp.float32),
    )(x2, y2)

    rec_loss = sse[0, 0] / x.size
    attn_loss = jnp.sum(bot) / attn.size
    return rec_loss + _ALPHA * attn_loss
